# baseline (speedup 1.0000x reference)
import functools

import jax
import jax.numpy as jnp
from jax import lax
from jax.experimental import pallas as pl
from jax.experimental.pallas import tpu as pltpu
from jax.experimental.pallas import tpu_sc as plsc

_NB = 128
_NG = 128
_NW = 32


def _build_matrix(r, E, l, h):
    V, W = E.shape

    def body(r_ref, e_ref, l_ref, h_ref, o_ref):
        rr = r_ref[...]
        o_ref[...] = e_ref[...] + rr * l_ref[...] + (1.0 - rr) * h_ref[...]

    return pl.pallas_call(
        body,
        out_shape=jax.ShapeDtypeStruct((V, W), jnp.float32),
    )(r, E, l.reshape(1, W), h.reshape(1, W))


def kernel(idx, r, E, l, h):
    V, W = E.shape
    B, F = idx.shape
    assert B % _NB == 0 and _NB % _NG == 0
    nsteps = B // _NB
    nblocks = F * nsteps
    assert nblocks % (4 * _NW) == 0
    perw = nblocks // _NW
    gpb = _NB // _NG

    matrix = _build_matrix(r, E, l, h)
    idx_t = idx.T.astype(jnp.int32).reshape(_NW, perw * gpb, _NG)

    mesh = plsc.VectorSubcoreMesh(
        core_axis_name="core", subcore_axis_name="subcore"
    )

    @functools.partial(
        pl.kernel,
        out_type=jax.ShapeDtypeStruct((F, B, W), jnp.float32),
        mesh=mesh,
        scratch_types=[
            pltpu.VMEM_SHARED((V, W), jnp.float32),
            pltpu.VMEM((perw * gpb, _NG), jnp.int32),
            pltpu.VMEM((_NB, W), jnp.float32),
            pltpu.VMEM((_NB, W), jnp.float32),
            pltpu.VMEM((_NB, W), jnp.float32),
            pltpu.VMEM((_NB, W), jnp.float32),
            pltpu.SemaphoreType.DMA,
            pltpu.SemaphoreType.DMA,
            pltpu.SemaphoreType.DMA,
            pltpu.SemaphoreType.DMA,
            pltpu.SemaphoreType.DMA,
            pltpu.SemaphoreType.DMA,
            pltpu.SemaphoreType.DMA,
            pltpu.SemaphoreType.DMA,
        ],
    )
    def gather_k(x_hbm, i_hbm, o_hbm, tbl_sh, idx_v, b0, b1, b2, b3,
                 g0, g1, g2, g3, o0, o1, o2, o3):
        cid = lax.axis_index("core")
        sid = lax.axis_index("subcore")
        wid = sid * 2 + cid

        @pl.when(sid == 0)
        def _():
            pltpu.sync_copy(x_hbm, tbl_sh)

        pltpu.sync_copy(i_hbm.at[wid], idx_v)
        plsc.subcore_barrier()

        base = wid * perw

        def run_block(kl, buf, gsem, osem, first):
            @pl.when(jnp.logical_not(first))
            def _():
                pltpu.make_async_copy(
                    buf, o_hbm.at[0, pl.ds(0, _NB)], osem
                ).wait()

            copies = [
                pltpu.async_copy(
                    tbl_sh.at[idx_v.at[kl * gpb + j]],
                    buf.at[pl.ds(j * _NG, _NG)],
                    gsem,
                )
                for j in range(gpb)
            ]
            for c in copies:
                c.wait()
            k = base + kl
            f = k // nsteps
            j = k % nsteps
            pltpu.async_copy(buf, o_hbm.at[f, pl.ds(j * _NB, _NB)], osem)

        @pl.loop(0, perw // 4)
        def _(t):
            run_block(4 * t, b0, g0, o0, t == 0)
            run_block(4 * t + 1, b1, g1, o1, t == 0)
            run_block(4 * t + 2, b2, g2, o2, t == 0)
            run_block(4 * t + 3, b3, g3, o3, t == 0)

        pltpu.make_async_copy(b0, o_hbm.at[0, pl.ds(0, _NB)], o0).wait()
        pltpu.make_async_copy(b1, o_hbm.at[0, pl.ds(0, _NB)], o1).wait()
        pltpu.make_async_copy(b2, o_hbm.at[0, pl.ds(0, _NB)], o2).wait()
        pltpu.make_async_copy(b3, o_hbm.at[0, pl.ds(0, _NB)], o3).wait()

    out_fbw = gather_k(matrix, idx_t)
    return out_fbw.transpose(1, 0, 2)


# speedup vs baseline: 1.0627x; 1.0627x over previous
import functools

import jax
import jax.numpy as jnp
from jax.experimental import pallas as pl
from jax.experimental.pallas import tpu as pltpu
from jax.experimental.pallas import tpu_sc as plsc

_NB = 256
_NG = 128


def _build_matrix(r, E, l, h):
    V, W = E.shape

    def body(r_ref, e_ref, l_ref, h_ref, o_ref):
        rr = r_ref[...]
        o_ref[...] = e_ref[...] + rr * l_ref[...] + (1.0 - rr) * h_ref[...]

    return pl.pallas_call(
        body,
        out_shape=jax.ShapeDtypeStruct((V, W), jnp.float32),
    )(r, E, l.reshape(1, W), h.reshape(1, W))


def kernel(idx, r, E, l, h):
    V, W = E.shape
    B, F = idx.shape
    assert B % _NB == 0 and _NB % _NG == 0
    nsteps = B // _NB

    matrix = _build_matrix(r, E, l, h)
    idx_t = idx.T.astype(jnp.int32)

    mesh = plsc.VectorSubcoreMesh(
        core_axis_name="core", subcore_axis_name="subcore"
    )

    @functools.partial(
        pl.kernel,
        out_type=jax.ShapeDtypeStruct((F, B, W), jnp.float32),
        mesh=mesh,
        scratch_types=[
            pltpu.VMEM_SHARED((V, W), jnp.float32),
            pltpu.SemaphoreType.DMA,
            pltpu.SemaphoreType.DMA,
        ],
    )
    def gather_k(x_hbm, i_hbm, o_hbm, tbl_sh, s0, s1):
        @pl.when(jax.lax.axis_index("subcore") == 0)
        def _():
            pltpu.sync_copy(x_hbm, tbl_sh)

        plsc.subcore_barrier()

        sems = (s0, s1)

        def body(i_vmem, o_vmem):
            copies = [
                pltpu.async_copy(
                    tbl_sh.at[i_vmem.at[0, pl.ds(j * _NG, _NG)]],
                    o_vmem.at[0, pl.ds(j * _NG, _NG)],
                    sems[j],
                )
                for j in range(_NB // _NG)
            ]
            for c in copies:
                c.wait()

        pltpu.emit_pipeline(
            body,
            grid=(F * nsteps,),
            in_specs=[
                pl.BlockSpec(
                    (1, _NB), index_map=lambda i: (i // nsteps, i % nsteps)
                )
            ],
            out_specs=[
                pl.BlockSpec(
                    (1, _NB, W),
                    index_map=lambda i: (i // nsteps, i % nsteps, 0),
                )
            ],
            core_axis_name=("core", "subcore"),
            dimension_semantics=(pltpu.PARALLEL,),
        )(i_hbm, o_hbm)

    out_fbw = gather_k(matrix, idx_t)
    return out_fbw.transpose(1, 0, 2)
